# baseline (speedup 1.0000x reference)
import functools

import jax
import jax.numpy as jnp
from jax import lax
from jax.experimental import pallas as pl
from jax.experimental.pallas import tpu as pltpu
from jax.experimental.pallas import tpu_sc as plsc

N = 10000
E = 320000
D_IN = 128
H = 64
D_EDGE = 7

NC, NS = 2, 16
NW = NC * NS
CH = 128
KCH = 80
NBUF = 8
KB = KCH // NBUF
E_PAD = NW * CH * KCH
EPW = KCH * CH
N_PAD = 10240
ROWS = N_PAD // NS

BLK = 512

_EPS = 1e-5


def _ln(h, g, b):
    m = jnp.mean(h, axis=-1, keepdims=True)
    v = jnp.mean((h - m) * (h - m), axis=-1, keepdims=True)
    return (h - m) * jax.lax.rsqrt(v + _EPS) * g + b


def _node_enc_body(x_ref, w0_ref, b0_ref, wa_ref, wb_ref,
                   h_ref, p_ref, q_ref):
    h = jnp.dot(x_ref[...], w0_ref[...], preferred_element_type=jnp.float32)
    h = h + b0_ref[...]
    h_ref[...] = h
    p_ref[...] = jnp.dot(h, wa_ref[...], preferred_element_type=jnp.float32)
    q_ref[...] = jnp.dot(h, wb_ref[...], preferred_element_type=jnp.float32)


def _edge_enc_body(ea_ref, w2_ref, b3_ref, w4_ref, b5_ref, w6_ref, b7_ref,
                   g8_ref, b9_ref, wc1_ref, bc1_ref, wc2_ref, bc2_ref,
                   ec1_ref, ec2_ref):
    e = jnp.dot(ea_ref[...], w2_ref[...], preferred_element_type=jnp.float32)
    e = jnp.maximum(e + b3_ref[...], 0.0)
    e = jnp.dot(e, w4_ref[...], preferred_element_type=jnp.float32)
    e = jnp.maximum(e + b5_ref[...], 0.0)
    e = jnp.dot(e, w6_ref[...], preferred_element_type=jnp.float32) + b7_ref[...]
    e = _ln(e, g8_ref[...], b9_ref[...])
    ec1_ref[...] = jnp.dot(e, wc1_ref[...],
                           preferred_element_type=jnp.float32) + bc1_ref[...]
    ec2_ref[...] = jnp.dot(e, wc2_ref[...],
                           preferred_element_type=jnp.float32) + bc2_ref[...]


def _edge_msg_body(g_ref, ec_ref, w2_ref, b2_ref, ln_g_ref, ln_b_ref, m_ref):
    t = jnp.maximum(g_ref[...] + ec_ref[...], 0.0)
    m = jnp.dot(t, w2_ref[...], preferred_element_type=jnp.float32) + b2_ref[...]
    m_ref[...] = _ln(m, ln_g_ref[...], ln_b_ref[...])


def _node_upd_mid_body(h_ref, a0_ref, a1_ref, ua_ref, ub_ref, b6_ref,
                       w8_ref, b8_ref, ln_g_ref, ln_b_ref,
                       wa_ref, wb_ref, h_out_ref, p_ref, q_ref):
    h = h_ref[...]
    agg = a0_ref[...] + a1_ref[...]
    u = jnp.dot(h, ua_ref[...], preferred_element_type=jnp.float32)
    u = u + jnp.dot(agg, ub_ref[...], preferred_element_type=jnp.float32)
    u = jnp.maximum(u + b6_ref[...], 0.0)
    u = jnp.dot(u, w8_ref[...], preferred_element_type=jnp.float32) + b8_ref[...]
    hn = _ln(u, ln_g_ref[...], ln_b_ref[...]) + h
    h_out_ref[...] = hn
    p_ref[...] = jnp.dot(hn, wa_ref[...], preferred_element_type=jnp.float32)
    q_ref[...] = jnp.dot(hn, wb_ref[...], preferred_element_type=jnp.float32)


def _node_upd_fin_body(h_ref, a0_ref, a1_ref, ua_ref, ub_ref, b6_ref,
                       w8_ref, b8_ref, ln_g_ref, ln_b_ref,
                       wo_ref, bo_ref, o_ref):
    h = h_ref[...]
    agg = a0_ref[...] + a1_ref[...]
    u = jnp.dot(h, ua_ref[...], preferred_element_type=jnp.float32)
    u = u + jnp.dot(agg, ub_ref[...], preferred_element_type=jnp.float32)
    u = jnp.maximum(u + b6_ref[...], 0.0)
    u = jnp.dot(u, w8_ref[...], preferred_element_type=jnp.float32) + b8_ref[...]
    hn = _ln(u, ln_g_ref[...], ln_b_ref[...]) + h
    o_ref[...] = jnp.dot(hn, wo_ref[...],
                         preferred_element_type=jnp.float32) + bo_ref[...]


def _row_spec(nb=BLK, d=H):
    return pl.BlockSpec((nb, d), lambda i: (i, 0))


def _full_spec(shape):
    return pl.BlockSpec(shape, lambda i: tuple(0 for _ in shape))


def _tc_call(body, grid, in_specs, out_specs, out_shape):
    return pl.pallas_call(
        body, grid=(grid,), in_specs=in_specs, out_specs=out_specs,
        out_shape=out_shape)


_MESH = plsc.VectorSubcoreMesh(core_axis_name="c", subcore_axis_name="s",
                               num_cores=NC, num_subcores=NS)


@functools.partial(
    pl.kernel,
    out_type=jax.ShapeDtypeStruct((E_PAD, H), jnp.float32),
    mesh=_MESH,
    scratch_types=[
        pltpu.VMEM((KCH, CH), jnp.int32),
        pltpu.VMEM((KCH, CH), jnp.int32),
        pltpu.VMEM((NBUF, CH, H), jnp.float32),
        pltpu.SemaphoreType.DMA((NBUF,)),
        pltpu.SemaphoreType.DMA((NBUF,)),
        pltpu.SemaphoreType.DMA((NBUF,)),
    ],
    compiler_params=pltpu.CompilerParams(use_tc_tiling_on_sc=False),
)
def _sc_gather(p_hbm, q_hbm, src_hbm, dst_hbm, g_hbm, src_v, dst_v, buf,
               semp, semq, semw):
    wid = lax.axis_index("s") * NC + lax.axis_index("c")
    base = wid * EPW
    pltpu.sync_copy(src_hbm.at[wid], src_v)
    pltpu.sync_copy(dst_hbm.at[wid], dst_v)

    def body(kb, carry):
        for b in range(NBUF):
            @pl.when(kb > 0)
            def _():
                pltpu.make_async_copy(
                    buf.at[b], g_hbm.at[pl.ds(base, CH)], semw.at[b]).wait()
            pltpu.async_copy(p_hbm.at[src_v.at[kb * NBUF + b]], buf.at[b],
                             semp.at[b])
        for b in range(NBUF):
            j = kb * NBUF + b
            pltpu.make_async_copy(p_hbm.at[src_v.at[j]], buf.at[b],
                                  semp.at[b]).wait()
            pltpu.async_copy(q_hbm.at[dst_v.at[j]], buf.at[b], semq.at[b],
                             add=True)
        for b in range(NBUF):
            j = kb * NBUF + b
            pltpu.make_async_copy(q_hbm.at[dst_v.at[j]], buf.at[b],
                                  semq.at[b]).wait()
            pltpu.async_copy(buf.at[b], g_hbm.at[pl.ds(base + j * CH, CH)],
                             semw.at[b])
        return carry

    lax.fori_loop(0, KB, body, 0)
    for b in range(NBUF):
        pltpu.make_async_copy(
            buf.at[b], g_hbm.at[pl.ds(base, CH)], semw.at[b]).wait()


@functools.partial(
    pl.kernel,
    out_type=jax.ShapeDtypeStruct((NC, N_PAD, H), jnp.float32),
    mesh=_MESH,
    scratch_types=[
        pltpu.VMEM((KCH, CH), jnp.int32),
        pltpu.VMEM((NBUF, CH, H), jnp.float32),
        pltpu.VMEM_SHARED((N_PAD, H), jnp.float32),
        pltpu.SemaphoreType.DMA((NBUF,)),
        pltpu.SemaphoreType.DMA((NBUF,)),
    ],
    compiler_params=pltpu.CompilerParams(use_tc_tiling_on_sc=False),
)
def _sc_scatter(m_hbm, dst_hbm, zer_hbm, agg_hbm, dst_v, buf, shared,
                semm, sems):
    cid = lax.axis_index("c")
    sid = lax.axis_index("s")
    wid = sid * NC + cid
    base = wid * EPW
    RB = ROWS // CH
    for k in range(RB):
        pltpu.sync_copy(zer_hbm, buf.at[k])
        pltpu.sync_copy(buf.at[k], shared.at[pl.ds(sid * ROWS + k * CH, CH)])
    plsc.subcore_barrier()
    pltpu.sync_copy(dst_hbm.at[wid], dst_v)

    def body(kb, carry):
        for b in range(NBUF):
            j = kb * NBUF + b
            @pl.when(kb > 0)
            def _():
                pltpu.make_async_copy(buf.at[b], shared.at[dst_v.at[j]],
                                      sems.at[b]).wait()
            pltpu.async_copy(m_hbm.at[pl.ds(base + j * CH, CH)], buf.at[b],
                             semm.at[b])
        for b in range(NBUF):
            j = kb * NBUF + b
            pltpu.make_async_copy(m_hbm.at[pl.ds(base + j * CH, CH)],
                                  buf.at[b], semm.at[b]).wait()
            pltpu.async_copy(buf.at[b], shared.at[dst_v.at[j]], sems.at[b],
                             add=True)
        return carry

    lax.fori_loop(0, KB, body, 0)
    for b in range(NBUF):
        pltpu.make_async_copy(buf.at[b], shared.at[dst_v.at[0]],
                              sems.at[b]).wait()
    plsc.subcore_barrier()
    for k in range(RB):
        rows = pl.ds(sid * ROWS + k * CH, CH)
        pltpu.sync_copy(shared.at[rows], buf.at[k])
        pltpu.sync_copy(buf.at[k], agg_hbm.at[cid].at[rows])


def kernel(x, edge_index, edge_attr,
           w0, w1, w2, w3, w4, w5, w6, w7, w8, w9,
           w10, w11, w12, w13, w14, w15, w16, w17, w18, w19,
           w20, w21, w22, w23, w24, w25, w26, w27, w28, w29,
           w30, w31, w32, w33, w34, w35):
    f32 = jnp.float32
    x_p = jnp.zeros((N_PAD, D_IN), f32).at[:N].set(x)
    ea_p = jnp.zeros((E_PAD, 8), f32).at[:E, :D_EDGE].set(edge_attr)
    w2_p = jnp.zeros((8, H), f32).at[:D_EDGE].set(w2)
    src = edge_index[0]
    dst = edge_index[1]
    src_p = jnp.zeros((E_PAD,), jnp.int32).at[:E].set(src).reshape(NW, KCH, CH)
    dst_p = jnp.full((E_PAD,), N, jnp.int32).at[:E].set(dst).reshape(NW, KCH, CH)
    zer = jnp.zeros((CH, H), f32)
    w34_p = jnp.zeros((H, 8), f32).at[:, :3].set(w34)
    b35_p = jnp.zeros((1, 8), f32).at[:, :3].set(w35)

    def r2(v):
        return v.reshape(1, -1)

    wa1, wb1, wc1 = w10[:H], w10[H:2 * H], w10[2 * H:]
    wa2, wb2, wc2 = w22[:H], w22[H:2 * H], w22[2 * H:]
    ua1, ub1 = w16[:H], w16[H:]
    ua2, ub2 = w28[:H], w28[H:]

    gn = N_PAD // BLK
    ge = E_PAD // BLK

    h0, p1, q1 = _tc_call(
        _node_enc_body, gn,
        [pl.BlockSpec((BLK, D_IN), lambda i: (i, 0)),
         _full_spec((D_IN, H)), _full_spec((1, H)),
         _full_spec((H, H)), _full_spec((H, H))],
        [_row_spec(), _row_spec(), _row_spec()],
        [jax.ShapeDtypeStruct((N_PAD, H), f32)] * 3,
    )(x_p, w0, r2(w1), wa1, wb1)

    ec1, ec2 = _tc_call(
        _edge_enc_body, ge,
        [pl.BlockSpec((BLK, 8), lambda i: (i, 0)),
         _full_spec((8, H)), _full_spec((1, H)),
         _full_spec((H, H)), _full_spec((1, H)),
         _full_spec((H, H)), _full_spec((1, H)),
         _full_spec((1, H)), _full_spec((1, H)),
         _full_spec((H, H)), _full_spec((1, H)),
         _full_spec((H, H)), _full_spec((1, H))],
        [_row_spec(), _row_spec()],
        [jax.ShapeDtypeStruct((E_PAD, H), f32)] * 2,
    )(ea_p, w2_p, r2(w3), w4, r2(w5), w6, r2(w7), r2(w8), r2(w9),
      wc1, r2(w11), wc2, r2(w23))

    g1 = _sc_gather(p1, q1, src_p, dst_p)
    m1 = _tc_call(
        _edge_msg_body, ge,
        [_row_spec(), _row_spec(), _full_spec((H, H)), _full_spec((1, H)),
         _full_spec((1, H)), _full_spec((1, H))],
        _row_spec(),
        jax.ShapeDtypeStruct((E_PAD, H), f32),
    )(g1, ec1, w12, r2(w13), r2(w14), r2(w15))
    agg1 = _sc_scatter(m1, dst_p, zer)
    h1, p2, q2 = _tc_call(
        _node_upd_mid_body, gn,
        [_row_spec(), _row_spec(), _row_spec(),
         _full_spec((H, H)), _full_spec((H, H)), _full_spec((1, H)),
         _full_spec((H, H)), _full_spec((1, H)),
         _full_spec((1, H)), _full_spec((1, H)),
         _full_spec((H, H)), _full_spec((H, H))],
        [_row_spec(), _row_spec(), _row_spec()],
        [jax.ShapeDtypeStruct((N_PAD, H), f32)] * 3,
    )(h0, agg1[0], agg1[1], ua1, ub1, r2(w17), w18, r2(w19), r2(w20), r2(w21),
      wa2, wb2)

    g2 = _sc_gather(p2, q2, src_p, dst_p)
    m2 = _tc_call(
        _edge_msg_body, ge,
        [_row_spec(), _row_spec(), _full_spec((H, H)), _full_spec((1, H)),
         _full_spec((1, H)), _full_spec((1, H))],
        _row_spec(),
        jax.ShapeDtypeStruct((E_PAD, H), f32),
    )(g2, ec2, w24, r2(w25), r2(w26), r2(w27))
    agg2 = _sc_scatter(m2, dst_p, zer)
    out8 = _tc_call(
        _node_upd_fin_body, gn,
        [_row_spec(), _row_spec(), _row_spec(),
         _full_spec((H, H)), _full_spec((H, H)), _full_spec((1, H)),
         _full_spec((H, H)), _full_spec((1, H)),
         _full_spec((1, H)), _full_spec((1, H)),
         _full_spec((H, 8)), _full_spec((1, 8))],
        pl.BlockSpec((BLK, 8), lambda i: (i, 0)),
        jax.ShapeDtypeStruct((N_PAD, 8), f32),
    )(h1, agg2[0], agg2[1], ua2, ub2, r2(w29), w30, r2(w31), r2(w32), r2(w33),
      w34_p, b35_p)

    return out8[:N, :3]


# speedup vs baseline: 1.6635x; 1.6635x over previous
import functools

import jax
import jax.numpy as jnp
from jax import lax
from jax.experimental import pallas as pl
from jax.experimental.pallas import tpu as pltpu
from jax.experimental.pallas import tpu_sc as plsc

N = 10000
E = 320000
D_IN = 128
H = 64
D_EDGE = 7

NC, NS = 2, 16
NW = NC * NS
CH = 128
KCH = 80
NBUF = 8
KB = KCH // NBUF
E_PAD = NW * CH * KCH
EPW = KCH * CH
N_PAD = 10240
ROWS = N_PAD // NS

BLK = 512

_EPS = 1e-5


def _ln(h, g, b):
    m = jnp.mean(h, axis=-1, keepdims=True)
    v = jnp.mean((h - m) * (h - m), axis=-1, keepdims=True)
    return (h - m) * jax.lax.rsqrt(v + _EPS) * g + b


def _node_enc_body(x_ref, w0_ref, b0_ref, wa_ref, wb_ref,
                   h_ref, p_ref, q_ref):
    h = jnp.dot(x_ref[...], w0_ref[...], preferred_element_type=jnp.float32)
    h = h + b0_ref[...]
    h_ref[...] = h
    p_ref[...] = jnp.dot(h, wa_ref[...], preferred_element_type=jnp.float32)
    q_ref[...] = jnp.dot(h, wb_ref[...], preferred_element_type=jnp.float32)


def _ln_pair(e, mh_ref, g_ref, b_ref):
    mb = jnp.dot(e, mh_ref[...], preferred_element_type=jnp.float32)
    xc = e - mb
    vb = jnp.dot(xc * xc, mh_ref[...], preferred_element_type=jnp.float32)
    return xc * jax.lax.rsqrt(vb + _EPS) * g_ref[...] + b_ref[...]


def _edge_enc_body(eaL_ref, eaR_ref, w2_ref, b3_ref, w4_ref, b5_ref, w6_ref,
                   b7_ref, g8_ref, b9_ref, mh_ref, wc1_ref, bc1_ref, wc2_ref,
                   bc2_ref, ec1_ref, ec2_ref):
    eL = jnp.dot(eaL_ref[...], w2_ref[...], preferred_element_type=jnp.float32)
    eR = jnp.dot(eaR_ref[...], w2_ref[...], preferred_element_type=jnp.float32)
    e = jnp.concatenate([eL, eR], axis=1)
    e = jnp.maximum(e + b3_ref[...], 0.0)
    e = jnp.dot(e, w4_ref[...], preferred_element_type=jnp.float32)
    e = jnp.maximum(e + b5_ref[...], 0.0)
    e = jnp.dot(e, w6_ref[...], preferred_element_type=jnp.float32) + b7_ref[...]
    e = _ln_pair(e, mh_ref, g8_ref, b9_ref)
    ec1_ref[...] = jnp.dot(e, wc1_ref[...],
                           preferred_element_type=jnp.float32) + bc1_ref[...]
    ec2_ref[...] = jnp.dot(e, wc2_ref[...],
                           preferred_element_type=jnp.float32) + bc2_ref[...]


def _edge_msg_body(g_ref, ec_ref, mh_ref, w2_ref, b2_ref, ln_g_ref, ln_b_ref,
                   m_ref):
    t = jnp.maximum(g_ref[...] + ec_ref[...], 0.0)
    m = jnp.dot(t, w2_ref[...], preferred_element_type=jnp.float32) + b2_ref[...]
    m_ref[...] = _ln_pair(m, mh_ref, ln_g_ref, ln_b_ref)


def _node_upd_mid_body(h_ref, a0_ref, a1_ref, ua_ref, ub_ref, b6_ref,
                       w8_ref, b8_ref, ln_g_ref, ln_b_ref,
                       wa_ref, wb_ref, h_out_ref, p_ref, q_ref):
    h = h_ref[...]
    agg = a0_ref[...] + a1_ref[...]
    u = jnp.dot(h, ua_ref[...], preferred_element_type=jnp.float32)
    u = u + jnp.dot(agg, ub_ref[...], preferred_element_type=jnp.float32)
    u = jnp.maximum(u + b6_ref[...], 0.0)
    u = jnp.dot(u, w8_ref[...], preferred_element_type=jnp.float32) + b8_ref[...]
    hn = _ln(u, ln_g_ref[...], ln_b_ref[...]) + h
    h_out_ref[...] = hn
    p_ref[...] = jnp.dot(hn, wa_ref[...], preferred_element_type=jnp.float32)
    q_ref[...] = jnp.dot(hn, wb_ref[...], preferred_element_type=jnp.float32)


def _node_upd_fin_body(h_ref, a0_ref, a1_ref, ua_ref, ub_ref, b6_ref,
                       w8_ref, b8_ref, ln_g_ref, ln_b_ref,
                       wo_ref, bo_ref, o_ref):
    h = h_ref[...]
    agg = a0_ref[...] + a1_ref[...]
    u = jnp.dot(h, ua_ref[...], preferred_element_type=jnp.float32)
    u = u + jnp.dot(agg, ub_ref[...], preferred_element_type=jnp.float32)
    u = jnp.maximum(u + b6_ref[...], 0.0)
    u = jnp.dot(u, w8_ref[...], preferred_element_type=jnp.float32) + b8_ref[...]
    hn = _ln(u, ln_g_ref[...], ln_b_ref[...]) + h
    o_ref[...] = jnp.dot(hn, wo_ref[...],
                         preferred_element_type=jnp.float32) + bo_ref[...]


def _row_spec(nb=BLK, d=H):
    return pl.BlockSpec((nb, d), lambda i: (i, 0))


def _full_spec(shape):
    return pl.BlockSpec(shape, lambda i: tuple(0 for _ in shape))


def _tc_call(body, grid, in_specs, out_specs, out_shape):
    return pl.pallas_call(
        body, grid=(grid,), in_specs=in_specs, out_specs=out_specs,
        out_shape=out_shape)


_MESH = plsc.VectorSubcoreMesh(core_axis_name="c", subcore_axis_name="s",
                               num_cores=NC, num_subcores=NS)


@functools.partial(
    pl.kernel,
    out_type=jax.ShapeDtypeStruct((E_PAD, H), jnp.float32),
    mesh=_MESH,
    scratch_types=[
        pltpu.VMEM((KCH, CH), jnp.int32),
        pltpu.VMEM((KCH, CH), jnp.int32),
        pltpu.VMEM((NBUF, CH, H), jnp.float32),
        pltpu.SemaphoreType.DMA((NBUF,)),
        pltpu.SemaphoreType.DMA((NBUF,)),
        pltpu.SemaphoreType.DMA((NBUF,)),
    ],
    compiler_params=pltpu.CompilerParams(use_tc_tiling_on_sc=False),
)
def _sc_gather(p_hbm, q_hbm, src_hbm, dst_hbm, g_hbm, src_v, dst_v, buf,
               semp, semq, semw):
    wid = lax.axis_index("s") * NC + lax.axis_index("c")
    base = wid * EPW
    pltpu.sync_copy(src_hbm.at[wid], src_v)
    pltpu.sync_copy(dst_hbm.at[wid], dst_v)

    def body(kb, carry):
        for b in range(NBUF):
            @pl.when(kb > 0)
            def _():
                pltpu.make_async_copy(
                    buf.at[b], g_hbm.at[pl.ds(base, CH)], semw.at[b]).wait()
            pltpu.async_copy(p_hbm.at[src_v.at[kb * NBUF + b]], buf.at[b],
                             semp.at[b])
        for b in range(NBUF):
            j = kb * NBUF + b
            pltpu.make_async_copy(p_hbm.at[src_v.at[j]], buf.at[b],
                                  semp.at[b]).wait()
            pltpu.async_copy(q_hbm.at[dst_v.at[j]], buf.at[b], semq.at[b],
                             add=True)
        for b in range(NBUF):
            j = kb * NBUF + b
            pltpu.make_async_copy(q_hbm.at[dst_v.at[j]], buf.at[b],
                                  semq.at[b]).wait()
            pltpu.async_copy(buf.at[b], g_hbm.at[pl.ds(base + j * CH, CH)],
                             semw.at[b])
        return carry

    lax.fori_loop(0, KB, body, 0)
    for b in range(NBUF):
        pltpu.make_async_copy(
            buf.at[b], g_hbm.at[pl.ds(base, CH)], semw.at[b]).wait()


@functools.partial(
    pl.kernel,
    out_type=jax.ShapeDtypeStruct((NC, N_PAD, H), jnp.float32),
    mesh=_MESH,
    scratch_types=[
        pltpu.VMEM((KCH, CH), jnp.int32),
        pltpu.VMEM((NBUF, CH, H), jnp.float32),
        pltpu.VMEM_SHARED((N_PAD, H), jnp.float32),
        pltpu.SemaphoreType.DMA((NBUF,)),
        pltpu.SemaphoreType.DMA((NBUF,)),
    ],
    compiler_params=pltpu.CompilerParams(use_tc_tiling_on_sc=False),
)
def _sc_scatter(m_hbm, dst_hbm, zer_hbm, agg_hbm, dst_v, buf, shared,
                semm, sems):
    cid = lax.axis_index("c")
    sid = lax.axis_index("s")
    wid = sid * NC + cid
    base = wid * EPW
    RB = ROWS // CH
    for k in range(RB):
        pltpu.sync_copy(zer_hbm, buf.at[k])
        pltpu.sync_copy(buf.at[k], shared.at[pl.ds(sid * ROWS + k * CH, CH)])
    plsc.subcore_barrier()
    pltpu.sync_copy(dst_hbm.at[wid], dst_v)

    def body(kb, carry):
        for b in range(NBUF):
            j = kb * NBUF + b
            @pl.when(kb > 0)
            def _():
                pltpu.make_async_copy(buf.at[b], shared.at[dst_v.at[j]],
                                      sems.at[b]).wait()
            pltpu.async_copy(m_hbm.at[pl.ds(base + j * CH, CH)], buf.at[b],
                             semm.at[b])
        for b in range(NBUF):
            j = kb * NBUF + b
            pltpu.make_async_copy(m_hbm.at[pl.ds(base + j * CH, CH)],
                                  buf.at[b], semm.at[b]).wait()
            pltpu.async_copy(buf.at[b], shared.at[dst_v.at[j]], sems.at[b],
                             add=True)
        return carry

    lax.fori_loop(0, KB, body, 0)
    for b in range(NBUF):
        pltpu.make_async_copy(buf.at[b], shared.at[dst_v.at[0]],
                              sems.at[b]).wait()
    plsc.subcore_barrier()
    for k in range(RB):
        rows = pl.ds(sid * ROWS + k * CH, CH)
        pltpu.sync_copy(shared.at[rows], buf.at[k])
        pltpu.sync_copy(buf.at[k], agg_hbm.at[cid].at[rows])


def kernel(x, edge_index, edge_attr,
           w0, w1, w2, w3, w4, w5, w6, w7, w8, w9,
           w10, w11, w12, w13, w14, w15, w16, w17, w18, w19,
           w20, w21, w22, w23, w24, w25, w26, w27, w28, w29,
           w30, w31, w32, w33, w34, w35):
    f32 = jnp.float32
    x_p = jnp.zeros((N_PAD, D_IN), f32).at[:N].set(x)
    src = edge_index[0]
    dst = edge_index[1]

    def il(v, fill):
        vp = jnp.full((E_PAD,), fill, jnp.int32).at[:E].set(v)
        return jnp.stack([vp[:E_PAD // 2], vp[E_PAD // 2:]], axis=1).reshape(-1)

    src_p = il(src, 0).reshape(NW, KCH, CH)
    dst_p = il(dst, N).reshape(NW, KCH, CH)
    zer = jnp.zeros((CH, H), f32)
    w34_p = jnp.zeros((H, 8), f32).at[:, :3].set(w34)
    b35_p = jnp.zeros((1, 8), f32).at[:, :3].set(w35)

    def r2(v):
        return v.reshape(1, -1)

    def r2d(v):
        return jnp.concatenate([v, v]).reshape(1, 2 * H)

    zH = jnp.zeros((H, H), f32)

    def bd(w):
        return jnp.concatenate(
            [jnp.concatenate([w, zH], axis=1),
             jnp.concatenate([zH, w], axis=1)], axis=0)

    mh = bd(jnp.full((H, H), 1.0 / H, f32))

    wa1, wb1, wc1 = w10[:H], w10[H:2 * H], w10[2 * H:]
    wa2, wb2, wc2 = w22[:H], w22[H:2 * H], w22[2 * H:]
    ua1, ub1 = w16[:H], w16[H:]
    ua2, ub2 = w28[:H], w28[H:]

    gn = N_PAD // BLK
    gep = (E_PAD // 2) // BLK
    ROFF = (E_PAD // 2) // BLK
    RMAX = (E - E_PAD // 2) // BLK - 1
    pair_spec = pl.BlockSpec((BLK, 2 * H), lambda i: (i, 0))

    h0, p1, q1 = _tc_call(
        _node_enc_body, gn,
        [pl.BlockSpec((BLK, D_IN), lambda i: (i, 0)),
         _full_spec((D_IN, H)), _full_spec((1, H)),
         _full_spec((H, H)), _full_spec((H, H))],
        [_row_spec(), _row_spec(), _row_spec()],
        [jax.ShapeDtypeStruct((N_PAD, H), f32)] * 3,
    )(x_p, w0, r2(w1), wa1, wb1)

    ec1, ec2 = _tc_call(
        _edge_enc_body, gep,
        [pl.BlockSpec((BLK, D_EDGE), lambda i: (i, 0)),
         pl.BlockSpec((BLK, D_EDGE),
                      lambda i: (ROFF + jnp.minimum(i, RMAX), 0)),
         _full_spec((D_EDGE, H)), _full_spec((1, 2 * H)),
         _full_spec((2 * H, 2 * H)), _full_spec((1, 2 * H)),
         _full_spec((2 * H, 2 * H)), _full_spec((1, 2 * H)),
         _full_spec((1, 2 * H)), _full_spec((1, 2 * H)),
         _full_spec((2 * H, 2 * H)),
         _full_spec((2 * H, 2 * H)), _full_spec((1, 2 * H)),
         _full_spec((2 * H, 2 * H)), _full_spec((1, 2 * H))],
        [pair_spec, pair_spec],
        [jax.ShapeDtypeStruct((E_PAD // 2, 2 * H), f32)] * 2,
    )(edge_attr, edge_attr, w2, r2d(w3), bd(w4), r2d(w5), bd(w6), r2d(w7),
      r2d(w8), r2d(w9), mh, bd(wc1), r2d(w11), bd(wc2), r2d(w23))

    def edge_msg(g_flat, ec, w2_, b2, ln_g, ln_b):
        gp = g_flat.reshape(E_PAD // 2, 2 * H)
        mp = _tc_call(
            _edge_msg_body, gep,
            [pair_spec, pair_spec, _full_spec((2 * H, 2 * H)),
             _full_spec((2 * H, 2 * H)), _full_spec((1, 2 * H)),
             _full_spec((1, 2 * H)), _full_spec((1, 2 * H))],
            pair_spec,
            jax.ShapeDtypeStruct((E_PAD // 2, 2 * H), f32),
        )(gp, ec, mh, bd(w2_), r2d(b2), r2d(ln_g), r2d(ln_b))
        return mp.reshape(E_PAD, H)

    g1 = _sc_gather(p1, q1, src_p, dst_p)
    m1 = edge_msg(g1, ec1, w12, w13, w14, w15)
    agg1 = _sc_scatter(m1, dst_p, zer)
    h1, p2, q2 = _tc_call(
        _node_upd_mid_body, gn,
        [_row_spec(), _row_spec(), _row_spec(),
         _full_spec((H, H)), _full_spec((H, H)), _full_spec((1, H)),
         _full_spec((H, H)), _full_spec((1, H)),
         _full_spec((1, H)), _full_spec((1, H)),
         _full_spec((H, H)), _full_spec((H, H))],
        [_row_spec(), _row_spec(), _row_spec()],
        [jax.ShapeDtypeStruct((N_PAD, H), f32)] * 3,
    )(h0, agg1[0], agg1[1], ua1, ub1, r2(w17), w18, r2(w19), r2(w20), r2(w21),
      wa2, wb2)

    g2 = _sc_gather(p2, q2, src_p, dst_p)
    m2 = edge_msg(g2, ec2, w24, w25, w26, w27)
    agg2 = _sc_scatter(m2, dst_p, zer)
    out8 = _tc_call(
        _node_upd_fin_body, gn,
        [_row_spec(), _row_spec(), _row_spec(),
         _full_spec((H, H)), _full_spec((H, H)), _full_spec((1, H)),
         _full_spec((H, H)), _full_spec((1, H)),
         _full_spec((1, H)), _full_spec((1, H)),
         _full_spec((H, 8)), _full_spec((1, 8))],
        pl.BlockSpec((BLK, 8), lambda i: (i, 0)),
        jax.ShapeDtypeStruct((N_PAD, 8), f32),
    )(h1, agg2[0], agg2[1], ua2, ub2, r2(w29), w30, r2(w31), r2(w32), r2(w33),
      w34_p, b35_p)

    return out8[:N, :3]


# speedup vs baseline: 1.7472x; 1.0503x over previous
import functools

import jax
import jax.numpy as jnp
from jax import lax
from jax.experimental import pallas as pl
from jax.experimental.pallas import tpu as pltpu
from jax.experimental.pallas import tpu_sc as plsc

N = 10000
E = 320000
D_IN = 128
H = 64
D_EDGE = 7

NC, NS = 2, 16
NW = NC * NS
CH = 128
KCH = 80
NBUF = 8
KB = KCH // NBUF
E_PAD = NW * CH * KCH
EPW = KCH * CH
N_PAD = 10240
ROWS = N_PAD // NS

BLK = 512

_EPS = 1e-5


def _ln(h, g, b):
    m = jnp.mean(h, axis=-1, keepdims=True)
    v = jnp.mean((h - m) * (h - m), axis=-1, keepdims=True)
    return (h - m) * jax.lax.rsqrt(v + _EPS) * g + b


def _node_enc_body(x_ref, w0_ref, b0_ref, wa_ref, wb_ref,
                   h_ref, p_ref, q_ref):
    h = jnp.dot(x_ref[...], w0_ref[...], preferred_element_type=jnp.float32)
    h = h + b0_ref[...]
    h_ref[...] = h
    p_ref[...] = jnp.dot(h, wa_ref[...], preferred_element_type=jnp.float32)
    q_ref[...] = jnp.dot(h, wb_ref[...], preferred_element_type=jnp.float32)


def _ln_pair(e, mh_ref, g_ref, b_ref):
    mb = jnp.dot(e, mh_ref[...], preferred_element_type=jnp.float32)
    xc = e - mb
    vb = jnp.dot(xc * xc, mh_ref[...], preferred_element_type=jnp.float32)
    return xc * jax.lax.rsqrt(vb + _EPS) * g_ref[...] + b_ref[...]


def _edge_enc_body(eaL_ref, eaR_ref, w2_ref, b3_ref, w4_ref, b5_ref, w6_ref,
                   b7_ref, g8_ref, b9_ref, mh_ref, wc1_ref, bc1_ref, wc2_ref,
                   bc2_ref, ec1_ref, ec2_ref):
    eL = jnp.dot(eaL_ref[...], w2_ref[...], preferred_element_type=jnp.float32)
    eR = jnp.dot(eaR_ref[...], w2_ref[...], preferred_element_type=jnp.float32)
    e = jnp.concatenate([eL, eR], axis=1)
    e = jnp.maximum(e + b3_ref[...], 0.0)
    e = jnp.dot(e, w4_ref[...], preferred_element_type=jnp.float32)
    e = jnp.maximum(e + b5_ref[...], 0.0)
    e = jnp.dot(e, w6_ref[...], preferred_element_type=jnp.float32) + b7_ref[...]
    e = _ln_pair(e, mh_ref, g8_ref, b9_ref)
    ec1_ref[...] = jnp.dot(e, wc1_ref[...],
                           preferred_element_type=jnp.float32) + bc1_ref[...]
    ec2_ref[...] = jnp.dot(e, wc2_ref[...],
                           preferred_element_type=jnp.float32) + bc2_ref[...]


def _edge_msg_body(g_ref, ec_ref, mh_ref, w2_ref, b2_ref, ln_g_ref, ln_b_ref,
                   m_ref):
    t = jnp.maximum(g_ref[...] + ec_ref[...], 0.0)
    m = jnp.dot(t, w2_ref[...], preferred_element_type=jnp.float32) + b2_ref[...]
    m_ref[...] = _ln_pair(m, mh_ref, ln_g_ref, ln_b_ref)


def _node_upd_mid_body(h_ref, a0_ref, a1_ref, ua_ref, ub_ref, b6_ref,
                       w8_ref, b8_ref, ln_g_ref, ln_b_ref,
                       wa_ref, wb_ref, h_out_ref, p_ref, q_ref):
    h = h_ref[...]
    agg = a0_ref[...] + a1_ref[...]
    u = jnp.dot(h, ua_ref[...], preferred_element_type=jnp.float32)
    u = u + jnp.dot(agg, ub_ref[...], preferred_element_type=jnp.float32)
    u = jnp.maximum(u + b6_ref[...], 0.0)
    u = jnp.dot(u, w8_ref[...], preferred_element_type=jnp.float32) + b8_ref[...]
    hn = _ln(u, ln_g_ref[...], ln_b_ref[...]) + h
    h_out_ref[...] = hn
    p_ref[...] = jnp.dot(hn, wa_ref[...], preferred_element_type=jnp.float32)
    q_ref[...] = jnp.dot(hn, wb_ref[...], preferred_element_type=jnp.float32)


def _node_upd_fin_body(h_ref, a0_ref, a1_ref, ua_ref, ub_ref, b6_ref,
                       w8_ref, b8_ref, ln_g_ref, ln_b_ref,
                       wo_ref, bo_ref, o_ref):
    h = h_ref[...]
    agg = a0_ref[...] + a1_ref[...]
    u = jnp.dot(h, ua_ref[...], preferred_element_type=jnp.float32)
    u = u + jnp.dot(agg, ub_ref[...], preferred_element_type=jnp.float32)
    u = jnp.maximum(u + b6_ref[...], 0.0)
    u = jnp.dot(u, w8_ref[...], preferred_element_type=jnp.float32) + b8_ref[...]
    hn = _ln(u, ln_g_ref[...], ln_b_ref[...]) + h
    o_ref[...] = jnp.dot(hn, wo_ref[...],
                         preferred_element_type=jnp.float32) + bo_ref[...]


def _row_spec(nb=BLK, d=H):
    return pl.BlockSpec((nb, d), lambda i: (i, 0))


def _full_spec(shape):
    return pl.BlockSpec(shape, lambda i: tuple(0 for _ in shape))


def _tc_call(body, grid, in_specs, out_specs, out_shape):
    return pl.pallas_call(
        body, grid=(grid,), in_specs=in_specs, out_specs=out_specs,
        out_shape=out_shape)


_MESH = plsc.VectorSubcoreMesh(core_axis_name="c", subcore_axis_name="s",
                               num_cores=NC, num_subcores=NS)


@functools.partial(
    pl.kernel,
    out_type=jax.ShapeDtypeStruct((E_PAD // 2, 2 * H), jnp.float32),
    mesh=_MESH,
    scratch_types=[
        pltpu.VMEM((KCH, CH), jnp.int32),
        pltpu.VMEM((KCH, CH), jnp.int32),
        pltpu.VMEM((NBUF, CH, H), jnp.float32),
        pltpu.SemaphoreType.DMA((NBUF,)),
        pltpu.SemaphoreType.DMA((NBUF,)),
        pltpu.SemaphoreType.DMA((NBUF,)),
    ],
    compiler_params=pltpu.CompilerParams(use_tc_tiling_on_sc=False),
)
def _sc_gather(p_hbm, q_hbm, src_hbm, dst_hbm, g_hbm, src_v, dst_v, buf,
               semp, semq, semw):
    wid = lax.axis_index("s") * NC + lax.axis_index("c")
    rbase = (wid % (NW // 2)) * EPW
    c0 = (wid // (NW // 2)) * H
    pltpu.sync_copy(src_hbm.at[wid], src_v)
    pltpu.sync_copy(dst_hbm.at[wid], dst_v)

    def gslice(j):
        return g_hbm.at[pl.ds(rbase + j * CH, CH), pl.ds(c0, H)]

    def body(kb, carry):
        for b in range(NBUF):
            @pl.when(kb > 0)
            def _():
                pltpu.make_async_copy(buf.at[b], gslice(0), semw.at[b]).wait()
            pltpu.async_copy(p_hbm.at[src_v.at[kb * NBUF + b]], buf.at[b],
                             semp.at[b])
        for b in range(NBUF):
            j = kb * NBUF + b
            pltpu.make_async_copy(p_hbm.at[src_v.at[j]], buf.at[b],
                                  semp.at[b]).wait()
            pltpu.async_copy(q_hbm.at[dst_v.at[j]], buf.at[b], semq.at[b],
                             add=True)
        for b in range(NBUF):
            j = kb * NBUF + b
            pltpu.make_async_copy(q_hbm.at[dst_v.at[j]], buf.at[b],
                                  semq.at[b]).wait()
            pltpu.async_copy(buf.at[b], gslice(j), semw.at[b])
        return carry

    lax.fori_loop(0, KB, body, 0)
    for b in range(NBUF):
        pltpu.make_async_copy(buf.at[b], gslice(0), semw.at[b]).wait()


@functools.partial(
    pl.kernel,
    out_type=jax.ShapeDtypeStruct((NC, N_PAD, H), jnp.float32),
    mesh=_MESH,
    scratch_types=[
        pltpu.VMEM((KCH, CH), jnp.int32),
        pltpu.VMEM((NBUF, CH, H), jnp.float32),
        pltpu.VMEM_SHARED((N_PAD, H), jnp.float32),
        pltpu.SemaphoreType.DMA((NBUF,)),
        pltpu.SemaphoreType.DMA((NBUF,)),
    ],
    compiler_params=pltpu.CompilerParams(use_tc_tiling_on_sc=False),
)
def _sc_scatter(m_hbm, dst_hbm, zer_hbm, agg_hbm, dst_v, buf, shared,
                semm, sems):
    cid = lax.axis_index("c")
    sid = lax.axis_index("s")
    wid = sid * NC + cid
    rbase = (wid % (NW // 2)) * EPW
    c0 = (wid // (NW // 2)) * H

    def mslice(j):
        return m_hbm.at[pl.ds(rbase + j * CH, CH), pl.ds(c0, H)]
    RB = ROWS // CH
    for k in range(RB):
        pltpu.sync_copy(zer_hbm, buf.at[k])
        pltpu.sync_copy(buf.at[k], shared.at[pl.ds(sid * ROWS + k * CH, CH)])
    plsc.subcore_barrier()
    pltpu.sync_copy(dst_hbm.at[wid], dst_v)

    def body(kb, carry):
        for b in range(NBUF):
            j = kb * NBUF + b
            @pl.when(kb > 0)
            def _():
                pltpu.make_async_copy(buf.at[b], shared.at[dst_v.at[j]],
                                      sems.at[b]).wait()
            pltpu.async_copy(mslice(j), buf.at[b], semm.at[b])
        for b in range(NBUF):
            j = kb * NBUF + b
            pltpu.make_async_copy(mslice(j), buf.at[b], semm.at[b]).wait()
            pltpu.async_copy(buf.at[b], shared.at[dst_v.at[j]], sems.at[b],
                             add=True)
        return carry

    lax.fori_loop(0, KB, body, 0)
    for b in range(NBUF):
        pltpu.make_async_copy(buf.at[b], shared.at[dst_v.at[0]],
                              sems.at[b]).wait()
    plsc.subcore_barrier()
    for k in range(RB):
        rows = pl.ds(sid * ROWS + k * CH, CH)
        pltpu.sync_copy(shared.at[rows], buf.at[k])
        pltpu.sync_copy(buf.at[k], agg_hbm.at[cid].at[rows])


def kernel(x, edge_index, edge_attr,
           w0, w1, w2, w3, w4, w5, w6, w7, w8, w9,
           w10, w11, w12, w13, w14, w15, w16, w17, w18, w19,
           w20, w21, w22, w23, w24, w25, w26, w27, w28, w29,
           w30, w31, w32, w33, w34, w35):
    f32 = jnp.float32
    x_p = jnp.zeros((N_PAD, D_IN), f32).at[:N].set(x)
    src = edge_index[0]
    dst = edge_index[1]

    src_p = (jnp.zeros((E_PAD,), jnp.int32).at[:E].set(src)
             .reshape(NW, KCH, CH))
    dst_p = (jnp.full((E_PAD,), N, jnp.int32).at[:E].set(dst)
             .reshape(NW, KCH, CH))
    zer = jnp.zeros((CH, H), f32)
    w34_p = jnp.zeros((H, 8), f32).at[:, :3].set(w34)
    b35_p = jnp.zeros((1, 8), f32).at[:, :3].set(w35)

    def r2(v):
        return v.reshape(1, -1)

    def r2d(v):
        return jnp.concatenate([v, v]).reshape(1, 2 * H)

    zH = jnp.zeros((H, H), f32)

    def bd(w):
        return jnp.concatenate(
            [jnp.concatenate([w, zH], axis=1),
             jnp.concatenate([zH, w], axis=1)], axis=0)

    mh = bd(jnp.full((H, H), 1.0 / H, f32))

    wa1, wb1, wc1 = w10[:H], w10[H:2 * H], w10[2 * H:]
    wa2, wb2, wc2 = w22[:H], w22[H:2 * H], w22[2 * H:]
    ua1, ub1 = w16[:H], w16[H:]
    ua2, ub2 = w28[:H], w28[H:]

    gn = N_PAD // BLK
    gep = (E_PAD // 2) // BLK
    ROFF = (E_PAD // 2) // BLK
    RMAX = (E - E_PAD // 2) // BLK - 1
    pair_spec = pl.BlockSpec((BLK, 2 * H), lambda i: (i, 0))

    h0, p1, q1 = _tc_call(
        _node_enc_body, gn,
        [pl.BlockSpec((BLK, D_IN), lambda i: (i, 0)),
         _full_spec((D_IN, H)), _full_spec((1, H)),
         _full_spec((H, H)), _full_spec((H, H))],
        [_row_spec(), _row_spec(), _row_spec()],
        [jax.ShapeDtypeStruct((N_PAD, H), f32)] * 3,
    )(x_p, w0, r2(w1), wa1, wb1)

    ec1, ec2 = _tc_call(
        _edge_enc_body, gep,
        [pl.BlockSpec((BLK, D_EDGE), lambda i: (i, 0)),
         pl.BlockSpec((BLK, D_EDGE),
                      lambda i: (ROFF + jnp.minimum(i, RMAX), 0)),
         _full_spec((D_EDGE, H)), _full_spec((1, 2 * H)),
         _full_spec((2 * H, 2 * H)), _full_spec((1, 2 * H)),
         _full_spec((2 * H, 2 * H)), _full_spec((1, 2 * H)),
         _full_spec((1, 2 * H)), _full_spec((1, 2 * H)),
         _full_spec((2 * H, 2 * H)),
         _full_spec((2 * H, 2 * H)), _full_spec((1, 2 * H)),
         _full_spec((2 * H, 2 * H)), _full_spec((1, 2 * H))],
        [pair_spec, pair_spec],
        [jax.ShapeDtypeStruct((E_PAD // 2, 2 * H), f32)] * 2,
    )(edge_attr, edge_attr, w2, r2d(w3), bd(w4), r2d(w5), bd(w6), r2d(w7),
      r2d(w8), r2d(w9), mh, bd(wc1), r2d(w11), bd(wc2), r2d(w23))

    def edge_msg(gp, ec, w2_, b2, ln_g, ln_b):
        return _tc_call(
            _edge_msg_body, gep,
            [pair_spec, pair_spec, _full_spec((2 * H, 2 * H)),
             _full_spec((2 * H, 2 * H)), _full_spec((1, 2 * H)),
             _full_spec((1, 2 * H)), _full_spec((1, 2 * H))],
            pair_spec,
            jax.ShapeDtypeStruct((E_PAD // 2, 2 * H), f32),
        )(gp, ec, mh, bd(w2_), r2d(b2), r2d(ln_g), r2d(ln_b))

    g1 = _sc_gather(p1, q1, src_p, dst_p)
    m1 = edge_msg(g1, ec1, w12, w13, w14, w15)
    agg1 = _sc_scatter(m1, dst_p, zer)
    h1, p2, q2 = _tc_call(
        _node_upd_mid_body, gn,
        [_row_spec(), _row_spec(), _row_spec(),
         _full_spec((H, H)), _full_spec((H, H)), _full_spec((1, H)),
         _full_spec((H, H)), _full_spec((1, H)),
         _full_spec((1, H)), _full_spec((1, H)),
         _full_spec((H, H)), _full_spec((H, H))],
        [_row_spec(), _row_spec(), _row_spec()],
        [jax.ShapeDtypeStruct((N_PAD, H), f32)] * 3,
    )(h0, agg1[0], agg1[1], ua1, ub1, r2(w17), w18, r2(w19), r2(w20), r2(w21),
      wa2, wb2)

    g2 = _sc_gather(p2, q2, src_p, dst_p)
    m2 = edge_msg(g2, ec2, w24, w25, w26, w27)
    agg2 = _sc_scatter(m2, dst_p, zer)
    out8 = _tc_call(
        _node_upd_fin_body, gn,
        [_row_spec(), _row_spec(), _row_spec(),
         _full_spec((H, H)), _full_spec((H, H)), _full_spec((1, H)),
         _full_spec((H, H)), _full_spec((1, H)),
         _full_spec((1, H)), _full_spec((1, H)),
         _full_spec((H, 8)), _full_spec((1, 8))],
        pl.BlockSpec((BLK, 8), lambda i: (i, 0)),
        jax.ShapeDtypeStruct((N_PAD, 8), f32),
    )(h1, agg2[0], agg2[1], ua2, ub2, r2(w29), w30, r2(w31), r2(w32), r2(w33),
      w34_p, b35_p)

    return out8[:N, :3]


# speedup vs baseline: 2.0885x; 1.1953x over previous
import functools

import jax
import jax.numpy as jnp
from jax import lax
from jax.experimental import pallas as pl
from jax.experimental.pallas import tpu as pltpu
from jax.experimental.pallas import tpu_sc as plsc

N = 10000
E = 320000
D_IN = 128
H = 64
D_EDGE = 7

NC, NS = 2, 16
NW = NC * NS
CH = 128
KCH = 80
NBUF = 8
KB = KCH // NBUF
E_PAD = NW * CH * KCH
EPW = KCH * CH
N_PAD = 10240
ROWS = N_PAD // NS

BLK = 512
BLK_E = 1024

_EPS = 1e-5
_BF = jnp.bfloat16


def _ln(h, g, b):
    m = jnp.mean(h, axis=-1, keepdims=True)
    v = jnp.mean((h - m) * (h - m), axis=-1, keepdims=True)
    return (h - m) * jax.lax.rsqrt(v + _EPS) * g + b


def _node_enc_body(x_ref, w0_ref, b0_ref, wa_ref, wb_ref,
                   h_ref, p_ref, q_ref):
    h = jnp.dot(x_ref[...], w0_ref[...], preferred_element_type=jnp.float32)
    h = h + b0_ref[...]
    h_ref[...] = h
    p_ref[...] = jnp.dot(h, wa_ref[...],
                         preferred_element_type=jnp.float32).astype(_BF)
    q_ref[...] = jnp.dot(h, wb_ref[...],
                         preferred_element_type=jnp.float32).astype(_BF)


def _ln_pair(e, mh_ref, g_ref, b_ref):
    mb = jnp.dot(e.astype(_BF), mh_ref[...],
                 preferred_element_type=jnp.float32)
    xc = e - mb
    vb = jnp.dot((xc * xc).astype(_BF), mh_ref[...],
                 preferred_element_type=jnp.float32)
    return xc * jax.lax.rsqrt(vb + _EPS) * g_ref[...] + b_ref[...]


def _edge_enc_body(eaL_ref, eaR_ref, w2_ref, b3_ref, w4_ref, b5_ref, w6_ref,
                   b7_ref, g8_ref, b9_ref, mh_ref, wc1_ref, bc1_ref, wc2_ref,
                   bc2_ref, ec1_ref, ec2_ref):
    eL = jnp.dot(eaL_ref[...].astype(_BF), w2_ref[...],
                 preferred_element_type=jnp.float32)
    eR = jnp.dot(eaR_ref[...].astype(_BF), w2_ref[...],
                 preferred_element_type=jnp.float32)
    e = jnp.concatenate([eL, eR], axis=1)
    e = jnp.maximum(e + b3_ref[...], 0.0)
    e = jnp.dot(e.astype(_BF), w4_ref[...], preferred_element_type=jnp.float32)
    e = jnp.maximum(e + b5_ref[...], 0.0)
    e = jnp.dot(e.astype(_BF), w6_ref[...],
                preferred_element_type=jnp.float32) + b7_ref[...]
    e = _ln_pair(e, mh_ref, g8_ref, b9_ref)
    ec1_ref[...] = (jnp.dot(e.astype(_BF), wc1_ref[...],
                            preferred_element_type=jnp.float32)
                    + bc1_ref[...]).astype(_BF)
    ec2_ref[...] = (jnp.dot(e.astype(_BF), wc2_ref[...],
                            preferred_element_type=jnp.float32)
                    + bc2_ref[...]).astype(_BF)


def _edge_msg_body(g_ref, ec_ref, mh_ref, w2_ref, b2_ref, ln_g_ref, ln_b_ref,
                   m_ref):
    t = jnp.maximum(g_ref[...].astype(jnp.float32)
                    + ec_ref[...].astype(jnp.float32), 0.0)
    m = jnp.dot(t.astype(_BF), w2_ref[...],
                preferred_element_type=jnp.float32) + b2_ref[...]
    m_ref[...] = _ln_pair(m, mh_ref, ln_g_ref, ln_b_ref)


def _node_upd_mid_body(h_ref, a0_ref, a1_ref, ua_ref, ub_ref, b6_ref,
                       w8_ref, b8_ref, ln_g_ref, ln_b_ref,
                       wa_ref, wb_ref, h_out_ref, p_ref, q_ref):
    h = h_ref[...]
    agg = a0_ref[...] + a1_ref[...]
    u = jnp.dot(h, ua_ref[...], preferred_element_type=jnp.float32)
    u = u + jnp.dot(agg, ub_ref[...], preferred_element_type=jnp.float32)
    u = jnp.maximum(u + b6_ref[...], 0.0)
    u = jnp.dot(u, w8_ref[...], preferred_element_type=jnp.float32) + b8_ref[...]
    hn = _ln(u, ln_g_ref[...], ln_b_ref[...]) + h
    h_out_ref[...] = hn
    p_ref[...] = jnp.dot(hn, wa_ref[...],
                         preferred_element_type=jnp.float32).astype(_BF)
    q_ref[...] = jnp.dot(hn, wb_ref[...],
                         preferred_element_type=jnp.float32).astype(_BF)


def _node_upd_fin_body(h_ref, a0_ref, a1_ref, ua_ref, ub_ref, b6_ref,
                       w8_ref, b8_ref, ln_g_ref, ln_b_ref,
                       wo_ref, bo_ref, o_ref):
    h = h_ref[...]
    agg = a0_ref[...] + a1_ref[...]
    u = jnp.dot(h, ua_ref[...], preferred_element_type=jnp.float32)
    u = u + jnp.dot(agg, ub_ref[...], preferred_element_type=jnp.float32)
    u = jnp.maximum(u + b6_ref[...], 0.0)
    u = jnp.dot(u, w8_ref[...], preferred_element_type=jnp.float32) + b8_ref[...]
    hn = _ln(u, ln_g_ref[...], ln_b_ref[...]) + h
    o_ref[...] = jnp.dot(hn, wo_ref[...],
                         preferred_element_type=jnp.float32) + bo_ref[...]


def _row_spec(nb=BLK, d=H):
    return pl.BlockSpec((nb, d), lambda i: (i, 0))


def _full_spec(shape):
    return pl.BlockSpec(shape, lambda i: tuple(0 for _ in shape))


def _tc_call(body, grid, in_specs, out_specs, out_shape):
    return pl.pallas_call(
        body, grid=(grid,), in_specs=in_specs, out_specs=out_specs,
        out_shape=out_shape)


_MESH = plsc.VectorSubcoreMesh(core_axis_name="c", subcore_axis_name="s",
                               num_cores=NC, num_subcores=NS)


@functools.partial(
    pl.kernel,
    out_type=jax.ShapeDtypeStruct((E_PAD // 2, 2 * H), jnp.bfloat16),
    mesh=_MESH,
    scratch_types=[
        pltpu.VMEM((KCH, CH), jnp.int32),
        pltpu.VMEM((KCH, CH), jnp.int32),
        pltpu.VMEM((NBUF, CH, H), jnp.bfloat16),
        pltpu.SemaphoreType.DMA((NBUF,)),
        pltpu.SemaphoreType.DMA((NBUF,)),
        pltpu.SemaphoreType.DMA((NBUF,)),
    ],
    compiler_params=pltpu.CompilerParams(use_tc_tiling_on_sc=False),
)
def _sc_gather(p_hbm, q_hbm, src_hbm, dst_hbm, g_hbm, src_v, dst_v, buf,
               semp, semq, semw):
    wid = lax.axis_index("s") * NC + lax.axis_index("c")
    rbase = (wid % (NW // 2)) * EPW
    c0 = (wid // (NW // 2)) * H
    pltpu.sync_copy(src_hbm.at[wid], src_v)
    pltpu.sync_copy(dst_hbm.at[wid], dst_v)

    def gslice(j):
        return g_hbm.at[pl.ds(rbase + j * CH, CH), pl.ds(c0, H)]

    def body(kb, carry):
        for b in range(NBUF):
            @pl.when(kb > 0)
            def _():
                pltpu.make_async_copy(buf.at[b], gslice(0), semw.at[b]).wait()
            pltpu.async_copy(p_hbm.at[src_v.at[kb * NBUF + b]], buf.at[b],
                             semp.at[b])
        for b in range(NBUF):
            j = kb * NBUF + b
            pltpu.make_async_copy(p_hbm.at[src_v.at[j]], buf.at[b],
                                  semp.at[b]).wait()
            pltpu.async_copy(q_hbm.at[dst_v.at[j]], buf.at[b], semq.at[b],
                             add=True)
        for b in range(NBUF):
            j = kb * NBUF + b
            pltpu.make_async_copy(q_hbm.at[dst_v.at[j]], buf.at[b],
                                  semq.at[b]).wait()
            pltpu.async_copy(buf.at[b], gslice(j), semw.at[b])
        return carry

    lax.fori_loop(0, KB, body, 0)
    for b in range(NBUF):
        pltpu.make_async_copy(buf.at[b], gslice(0), semw.at[b]).wait()


@functools.partial(
    pl.kernel,
    out_type=jax.ShapeDtypeStruct((NC, N_PAD, H), jnp.float32),
    mesh=_MESH,
    scratch_types=[
        pltpu.VMEM((KCH, CH), jnp.int32),
        pltpu.VMEM((NBUF, CH, H), jnp.float32),
        pltpu.VMEM_SHARED((N_PAD, H), jnp.float32),
        pltpu.SemaphoreType.DMA((NBUF,)),
        pltpu.SemaphoreType.DMA((NBUF,)),
    ],
    compiler_params=pltpu.CompilerParams(use_tc_tiling_on_sc=False),
)
def _sc_scatter(m_hbm, dst_hbm, zer_hbm, agg_hbm, dst_v, buf, shared,
                semm, sems):
    cid = lax.axis_index("c")
    sid = lax.axis_index("s")
    wid = sid * NC + cid
    rbase = (wid % (NW // 2)) * EPW
    c0 = (wid // (NW // 2)) * H

    def mslice(j):
        return m_hbm.at[pl.ds(rbase + j * CH, CH), pl.ds(c0, H)]
    RB = ROWS // CH
    for k in range(RB):
        pltpu.sync_copy(zer_hbm, buf.at[k])
        pltpu.sync_copy(buf.at[k], shared.at[pl.ds(sid * ROWS + k * CH, CH)])
    plsc.subcore_barrier()
    pltpu.sync_copy(dst_hbm.at[wid], dst_v)

    def body(kb, carry):
        for b in range(NBUF):
            j = kb * NBUF + b
            @pl.when(kb > 0)
            def _():
                pltpu.make_async_copy(buf.at[b], shared.at[dst_v.at[j]],
                                      sems.at[b]).wait()
            pltpu.async_copy(mslice(j), buf.at[b], semm.at[b])
        for b in range(NBUF):
            j = kb * NBUF + b
            pltpu.make_async_copy(mslice(j), buf.at[b], semm.at[b]).wait()
            pltpu.async_copy(buf.at[b], shared.at[dst_v.at[j]], sems.at[b],
                             add=True)
        return carry

    lax.fori_loop(0, KB, body, 0)
    for b in range(NBUF):
        pltpu.make_async_copy(buf.at[b], shared.at[dst_v.at[0]],
                              sems.at[b]).wait()
    plsc.subcore_barrier()
    for k in range(RB):
        rows = pl.ds(sid * ROWS + k * CH, CH)
        pltpu.sync_copy(shared.at[rows], buf.at[k])
        pltpu.sync_copy(buf.at[k], agg_hbm.at[cid].at[rows])


def kernel(x, edge_index, edge_attr,
           w0, w1, w2, w3, w4, w5, w6, w7, w8, w9,
           w10, w11, w12, w13, w14, w15, w16, w17, w18, w19,
           w20, w21, w22, w23, w24, w25, w26, w27, w28, w29,
           w30, w31, w32, w33, w34, w35):
    f32 = jnp.float32
    x_p = jnp.zeros((N_PAD, D_IN), f32).at[:N].set(x)
    src = edge_index[0]
    dst = edge_index[1]

    src_p = (jnp.zeros((E_PAD,), jnp.int32).at[:E].set(src)
             .reshape(NW, KCH, CH))
    dst_p = (jnp.full((E_PAD,), N, jnp.int32).at[:E].set(dst)
             .reshape(NW, KCH, CH))
    zer = jnp.zeros((CH, H), f32)
    w34_p = jnp.zeros((H, 8), f32).at[:, :3].set(w34)
    b35_p = jnp.zeros((1, 8), f32).at[:, :3].set(w35)

    def r2(v):
        return v.reshape(1, -1)

    def r2d(v):
        return jnp.concatenate([v, v]).reshape(1, 2 * H)

    zH = jnp.zeros((H, H), f32)

    def bd(w):
        return jnp.concatenate(
            [jnp.concatenate([w, zH], axis=1),
             jnp.concatenate([zH, w], axis=1)], axis=0)

    mh = bd(jnp.full((H, H), 1.0 / H, f32)).astype(_BF)

    wa1, wb1, wc1 = w10[:H], w10[H:2 * H], w10[2 * H:]
    wa2, wb2, wc2 = w22[:H], w22[H:2 * H], w22[2 * H:]
    ua1, ub1 = w16[:H], w16[H:]
    ua2, ub2 = w28[:H], w28[H:]

    gn = N_PAD // BLK
    gep = (E_PAD // 2) // BLK_E
    ROFF = (E_PAD // 2) // BLK_E
    RMAX = -(-(E - E_PAD // 2) // BLK_E) - 1
    pair_spec = pl.BlockSpec((BLK_E, 2 * H), lambda i: (i, 0))

    h0, p1, q1 = _tc_call(
        _node_enc_body, gn,
        [pl.BlockSpec((BLK, D_IN), lambda i: (i, 0)),
         _full_spec((D_IN, H)), _full_spec((1, H)),
         _full_spec((H, H)), _full_spec((H, H))],
        [_row_spec(), _row_spec(), _row_spec()],
        [jax.ShapeDtypeStruct((N_PAD, H), f32),
         jax.ShapeDtypeStruct((N_PAD, H), _BF),
         jax.ShapeDtypeStruct((N_PAD, H), _BF)],
    )(x_p, w0, r2(w1), wa1, wb1)

    ec1, ec2 = _tc_call(
        _edge_enc_body, gep,
        [pl.BlockSpec((BLK_E, D_EDGE), lambda i: (i, 0)),
         pl.BlockSpec((BLK_E, D_EDGE),
                      lambda i: (ROFF + jnp.minimum(i, RMAX), 0)),
         _full_spec((D_EDGE, H)), _full_spec((1, 2 * H)),
         _full_spec((2 * H, 2 * H)), _full_spec((1, 2 * H)),
         _full_spec((2 * H, 2 * H)), _full_spec((1, 2 * H)),
         _full_spec((1, 2 * H)), _full_spec((1, 2 * H)),
         _full_spec((2 * H, 2 * H)),
         _full_spec((2 * H, 2 * H)), _full_spec((1, 2 * H)),
         _full_spec((2 * H, 2 * H)), _full_spec((1, 2 * H))],
        [pair_spec, pair_spec],
        [jax.ShapeDtypeStruct((E_PAD // 2, 2 * H), _BF)] * 2,
    )(edge_attr, edge_attr, w2.astype(_BF), r2d(w3), bd(w4).astype(_BF),
      r2d(w5), bd(w6).astype(_BF), r2d(w7), r2d(w8), r2d(w9), mh,
      bd(wc1).astype(_BF), r2d(w11), bd(wc2).astype(_BF), r2d(w23))

    def edge_msg(gp, ec, w2_, b2, ln_g, ln_b):
        return _tc_call(
            _edge_msg_body, gep,
            [pair_spec, pair_spec, _full_spec((2 * H, 2 * H)),
             _full_spec((2 * H, 2 * H)), _full_spec((1, 2 * H)),
             _full_spec((1, 2 * H)), _full_spec((1, 2 * H))],
            pair_spec,
            jax.ShapeDtypeStruct((E_PAD // 2, 2 * H), f32),
        )(gp, ec, mh, bd(w2_).astype(_BF), r2d(b2), r2d(ln_g), r2d(ln_b))

    g1 = _sc_gather(p1, q1, src_p, dst_p)
    m1 = edge_msg(g1, ec1, w12, w13, w14, w15)
    agg1 = _sc_scatter(m1, dst_p, zer)
    h1, p2, q2 = _tc_call(
        _node_upd_mid_body, gn,
        [_row_spec(), _row_spec(), _row_spec(),
         _full_spec((H, H)), _full_spec((H, H)), _full_spec((1, H)),
         _full_spec((H, H)), _full_spec((1, H)),
         _full_spec((1, H)), _full_spec((1, H)),
         _full_spec((H, H)), _full_spec((H, H))],
        [_row_spec(), _row_spec(), _row_spec()],
        [jax.ShapeDtypeStruct((N_PAD, H), f32),
         jax.ShapeDtypeStruct((N_PAD, H), _BF),
         jax.ShapeDtypeStruct((N_PAD, H), _BF)],
    )(h0, agg1[0], agg1[1], ua1, ub1, r2(w17), w18, r2(w19), r2(w20), r2(w21),
      wa2, wb2)

    g2 = _sc_gather(p2, q2, src_p, dst_p)
    m2 = edge_msg(g2, ec2, w24, w25, w26, w27)
    agg2 = _sc_scatter(m2, dst_p, zer)
    out8 = _tc_call(
        _node_upd_fin_body, gn,
        [_row_spec(), _row_spec(), _row_spec(),
         _full_spec((H, H)), _full_spec((H, H)), _full_spec((1, H)),
         _full_spec((H, H)), _full_spec((1, H)),
         _full_spec((1, H)), _full_spec((1, H)),
         _full_spec((H, 8)), _full_spec((1, 8))],
        pl.BlockSpec((BLK, 8), lambda i: (i, 0)),
        jax.ShapeDtypeStruct((N_PAD, 8), f32),
    )(h1, agg2[0], agg2[1], ua2, ub2, r2(w29), w30, r2(w31), r2(w32), r2(w33),
      w34_p, b35_p)

    return out8[:N, :3]


# speedup vs baseline: 2.1360x; 1.0227x over previous
import functools

import jax
import jax.numpy as jnp
from jax import lax
from jax.experimental import pallas as pl
from jax.experimental.pallas import tpu as pltpu
from jax.experimental.pallas import tpu_sc as plsc

N = 10000
E = 320000
D_IN = 128
H = 64
D_EDGE = 7

NC, NS = 2, 16
NW = NC * NS
CH = 128
KCH = 80
NBUF = 8
KB = KCH // NBUF
E_PAD = NW * CH * KCH
EPW = KCH * CH
CTOT = E_PAD // CH
K0, K1 = 120, 40
N_PAD = 10240
ROWS = N_PAD // NS

BLK = 512
BLK_E = 1024

_EPS = 1e-5
_BF = jnp.bfloat16


def _ln(h, g, b):
    m = jnp.mean(h, axis=-1, keepdims=True)
    v = jnp.mean((h - m) * (h - m), axis=-1, keepdims=True)
    return (h - m) * jax.lax.rsqrt(v + _EPS) * g + b


def _node_enc_body(x_ref, w0_ref, b0_ref, wa_ref, wb_ref,
                   h_ref, p_ref, q_ref):
    h = jnp.dot(x_ref[...], w0_ref[...], preferred_element_type=jnp.float32)
    h = h + b0_ref[...]
    h_ref[...] = h
    p_ref[...] = jnp.dot(h, wa_ref[...],
                         preferred_element_type=jnp.float32).astype(_BF)
    q_ref[...] = jnp.dot(h, wb_ref[...],
                         preferred_element_type=jnp.float32).astype(_BF)


def _ln_pair(e, mh_ref, g_ref, b_ref):
    mb = jnp.dot(e.astype(_BF), mh_ref[...],
                 preferred_element_type=jnp.float32)
    xc = e - mb
    vb = jnp.dot((xc * xc).astype(_BF), mh_ref[...],
                 preferred_element_type=jnp.float32)
    return xc * jax.lax.rsqrt(vb + _EPS) * g_ref[...] + b_ref[...]


def _edge_enc_body(eaL_ref, eaR_ref, w2_ref, b3_ref, w4_ref, b5_ref, w6_ref,
                   b7_ref, g8_ref, b9_ref, mh_ref, wc1_ref, bc1_ref, wc2_ref,
                   bc2_ref, ec1_ref, ec2_ref):
    eL = jnp.dot(eaL_ref[...].astype(_BF), w2_ref[...],
                 preferred_element_type=jnp.float32)
    eR = jnp.dot(eaR_ref[...].astype(_BF), w2_ref[...],
                 preferred_element_type=jnp.float32)
    e = jnp.concatenate([eL, eR], axis=1)
    e = jnp.maximum(e + b3_ref[...], 0.0)
    e = jnp.dot(e.astype(_BF), w4_ref[...], preferred_element_type=jnp.float32)
    e = jnp.maximum(e + b5_ref[...], 0.0)
    e = jnp.dot(e.astype(_BF), w6_ref[...],
                preferred_element_type=jnp.float32) + b7_ref[...]
    e = _ln_pair(e, mh_ref, g8_ref, b9_ref)
    ec1_ref[...] = (jnp.dot(e.astype(_BF), wc1_ref[...],
                            preferred_element_type=jnp.float32)
                    + bc1_ref[...]).astype(_BF)
    ec2_ref[...] = (jnp.dot(e.astype(_BF), wc2_ref[...],
                            preferred_element_type=jnp.float32)
                    + bc2_ref[...]).astype(_BF)


def _edge_msg_body(g_ref, ec_ref, mh_ref, w2_ref, b2_ref, ln_g_ref, ln_b_ref,
                   m_ref):
    t = jnp.maximum(g_ref[...].astype(jnp.float32)
                    + ec_ref[...].astype(jnp.float32), 0.0)
    m = jnp.dot(t.astype(_BF), w2_ref[...],
                preferred_element_type=jnp.float32) + b2_ref[...]
    m_ref[...] = _ln_pair(m, mh_ref, ln_g_ref, ln_b_ref)


def _node_upd_mid_body(h_ref, a0_ref, a1_ref, ua_ref, ub_ref, b6_ref,
                       w8_ref, b8_ref, ln_g_ref, ln_b_ref,
                       wa_ref, wb_ref, h_out_ref, p_ref, q_ref):
    h = h_ref[...]
    agg = a0_ref[...] + a1_ref[...]
    u = jnp.dot(h, ua_ref[...], preferred_element_type=jnp.float32)
    u = u + jnp.dot(agg, ub_ref[...], preferred_element_type=jnp.float32)
    u = jnp.maximum(u + b6_ref[...], 0.0)
    u = jnp.dot(u, w8_ref[...], preferred_element_type=jnp.float32) + b8_ref[...]
    hn = _ln(u, ln_g_ref[...], ln_b_ref[...]) + h
    h_out_ref[...] = hn
    p_ref[...] = jnp.dot(hn, wa_ref[...],
                         preferred_element_type=jnp.float32).astype(_BF)
    q_ref[...] = jnp.dot(hn, wb_ref[...],
                         preferred_element_type=jnp.float32).astype(_BF)


def _node_upd_fin_body(h_ref, a0_ref, a1_ref, ua_ref, ub_ref, b6_ref,
                       w8_ref, b8_ref, ln_g_ref, ln_b_ref,
                       wo_ref, bo_ref, o_ref):
    h = h_ref[...]
    agg = a0_ref[...] + a1_ref[...]
    u = jnp.dot(h, ua_ref[...], preferred_element_type=jnp.float32)
    u = u + jnp.dot(agg, ub_ref[...], preferred_element_type=jnp.float32)
    u = jnp.maximum(u + b6_ref[...], 0.0)
    u = jnp.dot(u, w8_ref[...], preferred_element_type=jnp.float32) + b8_ref[...]
    hn = _ln(u, ln_g_ref[...], ln_b_ref[...]) + h
    o_ref[...] = jnp.dot(hn, wo_ref[...],
                         preferred_element_type=jnp.float32) + bo_ref[...]


def _row_spec(nb=BLK, d=H):
    return pl.BlockSpec((nb, d), lambda i: (i, 0))


def _full_spec(shape):
    return pl.BlockSpec(shape, lambda i: tuple(0 for _ in shape))


def _tc_call(body, grid, in_specs, out_specs, out_shape):
    return pl.pallas_call(
        body, grid=(grid,), in_specs=in_specs, out_specs=out_specs,
        out_shape=out_shape)


_MESH = plsc.VectorSubcoreMesh(core_axis_name="c", subcore_axis_name="s",
                               num_cores=NC, num_subcores=NS)


@functools.partial(
    pl.kernel,
    out_type=jax.ShapeDtypeStruct((E_PAD // 2, 2 * H), jnp.bfloat16),
    mesh=_MESH,
    scratch_types=[
        pltpu.VMEM((K0, CH), jnp.int32),
        pltpu.VMEM((K0, CH), jnp.int32),
        pltpu.VMEM((NBUF, CH, H), jnp.bfloat16),
        pltpu.SemaphoreType.DMA((NBUF,)),
        pltpu.SemaphoreType.DMA((NBUF,)),
        pltpu.SemaphoreType.DMA((NBUF,)),
    ],
    compiler_params=pltpu.CompilerParams(use_tc_tiling_on_sc=False),
)
def _sc_gather(p_hbm, q_hbm, src_hbm, dst_hbm, g_hbm, src_v, dst_v, buf,
               semp, semq, semw):
    cid = lax.axis_index("c")
    sid = lax.axis_index("s")

    def gslice(c):
        row0 = lax.rem(c, CTOT // 2) * CH
        col0 = (c // (CTOT // 2)) * H
        return g_hbm.at[pl.ds(row0, CH), pl.ds(col0, H)]

    def run(cbase, k_chunks):
        pltpu.sync_copy(src_hbm.at[pl.ds(cbase, k_chunks)],
                        src_v.at[pl.ds(0, k_chunks)])
        pltpu.sync_copy(dst_hbm.at[pl.ds(cbase, k_chunks)],
                        dst_v.at[pl.ds(0, k_chunks)])

        def body(kb, carry):
            for b in range(NBUF):
                @pl.when(kb > 0)
                def _():
                    pltpu.make_async_copy(buf.at[b], gslice(0),
                                          semw.at[b]).wait()
                pltpu.async_copy(p_hbm.at[src_v.at[kb * NBUF + b]], buf.at[b],
                                 semp.at[b])
            for b in range(NBUF):
                j = kb * NBUF + b
                pltpu.make_async_copy(p_hbm.at[src_v.at[j]], buf.at[b],
                                      semp.at[b]).wait()
                pltpu.async_copy(q_hbm.at[dst_v.at[j]], buf.at[b], semq.at[b],
                                 add=True)
            for b in range(NBUF):
                j = kb * NBUF + b
                pltpu.make_async_copy(q_hbm.at[dst_v.at[j]], buf.at[b],
                                      semq.at[b]).wait()
                pltpu.async_copy(buf.at[b], gslice(cbase + j), semw.at[b])
            return carry

        lax.fori_loop(0, k_chunks // NBUF, body, 0)
        for b in range(NBUF):
            pltpu.make_async_copy(buf.at[b], gslice(0), semw.at[b]).wait()

    @pl.when(cid == 0)
    def _():
        run(sid * K0, K0)

    @pl.when(cid == 1)
    def _():
        run(NS * K0 + sid * K1, K1)


@functools.partial(
    pl.kernel,
    out_type=jax.ShapeDtypeStruct((NC, N_PAD, H), jnp.float32),
    mesh=_MESH,
    scratch_types=[
        pltpu.VMEM((KCH, CH), jnp.int32),
        pltpu.VMEM((NBUF, CH, H), jnp.float32),
        pltpu.VMEM_SHARED((N_PAD, H), jnp.float32),
        pltpu.SemaphoreType.DMA((NBUF,)),
        pltpu.SemaphoreType.DMA((NBUF,)),
    ],
    compiler_params=pltpu.CompilerParams(use_tc_tiling_on_sc=False),
)
def _sc_scatter(m_hbm, dst_hbm, zer_hbm, agg_hbm, dst_v, buf, shared,
                semm, sems):
    cid = lax.axis_index("c")
    sid = lax.axis_index("s")
    wid = sid * NC + cid
    rbase = (wid % (NW // 2)) * EPW
    c0 = (wid // (NW // 2)) * H

    def mslice(j):
        return m_hbm.at[pl.ds(rbase + j * CH, CH), pl.ds(c0, H)]
    RB = ROWS // CH
    for k in range(RB):
        pltpu.sync_copy(zer_hbm, buf.at[k])
        pltpu.sync_copy(buf.at[k], shared.at[pl.ds(sid * ROWS + k * CH, CH)])
    plsc.subcore_barrier()
    pltpu.sync_copy(dst_hbm.at[wid], dst_v)

    def body(kb, carry):
        for b in range(NBUF):
            j = kb * NBUF + b
            @pl.when(kb > 0)
            def _():
                pltpu.make_async_copy(buf.at[b], shared.at[dst_v.at[j]],
                                      sems.at[b]).wait()
            pltpu.async_copy(mslice(j), buf.at[b], semm.at[b])
        for b in range(NBUF):
            j = kb * NBUF + b
            pltpu.make_async_copy(mslice(j), buf.at[b], semm.at[b]).wait()
            pltpu.async_copy(buf.at[b], shared.at[dst_v.at[j]], sems.at[b],
                             add=True)
        return carry

    lax.fori_loop(0, KB, body, 0)
    for b in range(NBUF):
        pltpu.make_async_copy(buf.at[b], shared.at[dst_v.at[0]],
                              sems.at[b]).wait()
    plsc.subcore_barrier()
    for k in range(RB):
        rows = pl.ds(sid * ROWS + k * CH, CH)
        pltpu.sync_copy(shared.at[rows], buf.at[k])
        pltpu.sync_copy(buf.at[k], agg_hbm.at[cid].at[rows])


def kernel(x, edge_index, edge_attr,
           w0, w1, w2, w3, w4, w5, w6, w7, w8, w9,
           w10, w11, w12, w13, w14, w15, w16, w17, w18, w19,
           w20, w21, w22, w23, w24, w25, w26, w27, w28, w29,
           w30, w31, w32, w33, w34, w35):
    f32 = jnp.float32
    x_p = jnp.zeros((N_PAD, D_IN), f32).at[:N].set(x)
    src = edge_index[0]
    dst = edge_index[1]

    src_f = (jnp.zeros((E_PAD,), jnp.int32).at[:E].set(src)
             .reshape(CTOT, CH))
    dst_f = (jnp.full((E_PAD,), N, jnp.int32).at[:E].set(dst)
             .reshape(CTOT, CH))
    dst_p = dst_f.reshape(NW, KCH, CH)
    zer = jnp.zeros((CH, H), f32)
    w34_p = jnp.zeros((H, 8), f32).at[:, :3].set(w34)
    b35_p = jnp.zeros((1, 8), f32).at[:, :3].set(w35)

    def r2(v):
        return v.reshape(1, -1)

    def r2d(v):
        return jnp.concatenate([v, v]).reshape(1, 2 * H)

    zH = jnp.zeros((H, H), f32)

    def bd(w):
        return jnp.concatenate(
            [jnp.concatenate([w, zH], axis=1),
             jnp.concatenate([zH, w], axis=1)], axis=0)

    mh = bd(jnp.full((H, H), 1.0 / H, f32)).astype(_BF)

    wa1, wb1, wc1 = w10[:H], w10[H:2 * H], w10[2 * H:]
    wa2, wb2, wc2 = w22[:H], w22[H:2 * H], w22[2 * H:]
    ua1, ub1 = w16[:H], w16[H:]
    ua2, ub2 = w28[:H], w28[H:]

    gn = N_PAD // BLK
    gep = (E_PAD // 2) // BLK_E
    ROFF = (E_PAD // 2) // BLK_E
    RMAX = -(-(E - E_PAD // 2) // BLK_E) - 1
    pair_spec = pl.BlockSpec((BLK_E, 2 * H), lambda i: (i, 0))

    h0, p1, q1 = _tc_call(
        _node_enc_body, gn,
        [pl.BlockSpec((BLK, D_IN), lambda i: (i, 0)),
         _full_spec((D_IN, H)), _full_spec((1, H)),
         _full_spec((H, H)), _full_spec((H, H))],
        [_row_spec(), _row_spec(), _row_spec()],
        [jax.ShapeDtypeStruct((N_PAD, H), f32),
         jax.ShapeDtypeStruct((N_PAD, H), _BF),
         jax.ShapeDtypeStruct((N_PAD, H), _BF)],
    )(x_p, w0, r2(w1), wa1, wb1)

    ec1, ec2 = _tc_call(
        _edge_enc_body, gep,
        [pl.BlockSpec((BLK_E, D_EDGE), lambda i: (i, 0)),
         pl.BlockSpec((BLK_E, D_EDGE),
                      lambda i: (ROFF + jnp.minimum(i, RMAX), 0)),
         _full_spec((D_EDGE, H)), _full_spec((1, 2 * H)),
         _full_spec((2 * H, 2 * H)), _full_spec((1, 2 * H)),
         _full_spec((2 * H, 2 * H)), _full_spec((1, 2 * H)),
         _full_spec((1, 2 * H)), _full_spec((1, 2 * H)),
         _full_spec((2 * H, 2 * H)),
         _full_spec((2 * H, 2 * H)), _full_spec((1, 2 * H)),
         _full_spec((2 * H, 2 * H)), _full_spec((1, 2 * H))],
        [pair_spec, pair_spec],
        [jax.ShapeDtypeStruct((E_PAD // 2, 2 * H), _BF)] * 2,
    )(edge_attr, edge_attr, w2.astype(_BF), r2d(w3), bd(w4).astype(_BF),
      r2d(w5), bd(w6).astype(_BF), r2d(w7), r2d(w8), r2d(w9), mh,
      bd(wc1).astype(_BF), r2d(w11), bd(wc2).astype(_BF), r2d(w23))

    def edge_msg(gp, ec, w2_, b2, ln_g, ln_b):
        return _tc_call(
            _edge_msg_body, gep,
            [pair_spec, pair_spec, _full_spec((2 * H, 2 * H)),
             _full_spec((2 * H, 2 * H)), _full_spec((1, 2 * H)),
             _full_spec((1, 2 * H)), _full_spec((1, 2 * H))],
            pair_spec,
            jax.ShapeDtypeStruct((E_PAD // 2, 2 * H), f32),
        )(gp, ec, mh, bd(w2_).astype(_BF), r2d(b2), r2d(ln_g), r2d(ln_b))

    g1 = _sc_gather(p1, q1, src_f, dst_f)
    m1 = edge_msg(g1, ec1, w12, w13, w14, w15)
    agg1 = _sc_scatter(m1, dst_p, zer)
    h1, p2, q2 = _tc_call(
        _node_upd_mid_body, gn,
        [_row_spec(), _row_spec(), _row_spec(),
         _full_spec((H, H)), _full_spec((H, H)), _full_spec((1, H)),
         _full_spec((H, H)), _full_spec((1, H)),
         _full_spec((1, H)), _full_spec((1, H)),
         _full_spec((H, H)), _full_spec((H, H))],
        [_row_spec(), _row_spec(), _row_spec()],
        [jax.ShapeDtypeStruct((N_PAD, H), f32),
         jax.ShapeDtypeStruct((N_PAD, H), _BF),
         jax.ShapeDtypeStruct((N_PAD, H), _BF)],
    )(h0, agg1[0], agg1[1], ua1, ub1, r2(w17), w18, r2(w19), r2(w20), r2(w21),
      wa2, wb2)

    g2 = _sc_gather(p2, q2, src_f, dst_f)
    m2 = edge_msg(g2, ec2, w24, w25, w26, w27)
    agg2 = _sc_scatter(m2, dst_p, zer)
    out8 = _tc_call(
        _node_upd_fin_body, gn,
        [_row_spec(), _row_spec(), _row_spec(),
         _full_spec((H, H)), _full_spec((H, H)), _full_spec((1, H)),
         _full_spec((H, H)), _full_spec((1, H)),
         _full_spec((1, H)), _full_spec((1, H)),
         _full_spec((H, 8)), _full_spec((1, 8))],
        pl.BlockSpec((BLK, 8), lambda i: (i, 0)),
        jax.ShapeDtypeStruct((N_PAD, 8), f32),
    )(h1, agg2[0], agg2[1], ua2, ub2, r2(w29), w30, r2(w31), r2(w32), r2(w33),
      w34_p, b35_p)

    return out8[:N, :3]
